# baseline (speedup 1.0000x reference)
import functools

import jax
import jax.numpy as jnp
from jax import lax
from jax.experimental import pallas as pl
from jax.experimental.pallas import tpu as pltpu
from jax.experimental.pallas import tpu_sc as plsc

B = 16384
V = 1_000_000
D = 64
BLK = 2048
NEG_SLOPE = 0.01

NW = 32
RANGE = V // NW
SLAB = 768
NSLAB = 42
CAP = 48
LAST_LO = V - 64
MAXOFF = 999168


def _make_sc_gather():
    info = plsc.get_sparse_core_info()
    nc, ns = info.num_cores, info.num_subcores
    mesh = plsc.VectorSubcoreMesh(core_axis_name="c", subcore_axis_name="s")

    @functools.partial(
        pl.kernel,
        mesh=mesh,
        out_type=jax.ShapeDtypeStruct((B, 128), jnp.float32),
        scratch_types=[
            pltpu.VMEM((B,), jnp.int32),
            pltpu.VMEM((D, SLAB), jnp.float32),
            pltpu.VMEM((D, SLAB), jnp.float32),
            pltpu.VMEM((D, 64), jnp.float32),
            pltpu.VMEM((CAP, 128), jnp.float32),
            pltpu.VMEM((CAP,), jnp.int32),
            pltpu.VMEM((288,), jnp.int32),
            pltpu.SemaphoreType.DMA,
            pltpu.SemaphoreType.DMA,
            pltpu.SemaphoreType.DMA,
        ],
        compiler_params=pltpu.CompilerParams(needs_layout_passes=False),
    )
    def gather_sweep(tableT_hbm, tail_hbm, idx_hbm, out_hbm,
                     uids_v, slab_a, slab_b, tail_v, rows_v, pos_s, occ_v,
                     sem_a, sem_b, sem_o):
        wid = lax.axis_index("s") * nc + lax.axis_index("c")
        mylo = wid * RANGE
        myhi = mylo + RANGE
        start = (mylo // 128) * 128

        def _off_c(s):
            off = start + s * SLAB
            return pl.multiple_of(jnp.minimum(off, MAXOFF), 128)

        def stage_full(s, buf, sem_x):
            pltpu.make_async_copy(
                tableT_hbm.at[:, pl.ds(_off_c(s), SLAB)], buf, sem_x
            ).start()

        def stage(s, buf, sem_x):
            off_c = _off_c(s)
            flags = occ_v[pl.ds((off_c - start) // 128, 16)]
            for k in range(6):
                @pl.when(flags[k] > 0)
                def _fire(k=k):
                    src = tableT_hbm.at[
                        :, pl.ds(pl.multiple_of(off_c + k * 128, 128), 128)]
                    pltpu.make_async_copy(
                        src, buf.at[:, pl.ds(k * 128, 128)], sem_x
                    ).start()

        def wait_slab(s, buf, sem_x):
            @pl.when(s < 2)
            def _wfull():
                pltpu.make_async_copy(
                    tableT_hbm.at[:, pl.ds(0, SLAB)], buf, sem_x
                ).wait()

            @pl.when(s >= 2)
            def _wgroups():
                flags = occ_v[pl.ds((_off_c(s) - start) // 128, 16)]
                for k in range(6):
                    @pl.when(flags[k] > 0)
                    def _wk(k=k):
                        pltpu.make_async_copy(
                            tableT_hbm.at[:, pl.ds(0, 128)],
                            buf.at[:, pl.ds(k * 128, 128)], sem_x
                        ).wait()

        stage_full(0, slab_a, sem_a)
        stage_full(1, slab_b, sem_b)
        pltpu.sync_copy(idx_hbm, uids_v)
        pltpu.sync_copy(tail_hbm, tail_v)

        neg1 = jnp.full((16,), -1, jnp.int32)
        sentinel = jnp.full((16,), 0x7FFFFFFF, jnp.int32)
        iota16 = lax.iota(jnp.int32, 16)

        def prefill_pos(i, c):
            pos_s[pl.ds(i * 16, 16)] = neg1
            return c

        lax.fori_loop(0, CAP // 16, prefill_pos, 0)

        zero16 = jnp.zeros((16,), jnp.int32)

        def occ_zero(i, c):
            occ_v[pl.ds(i * 16, 16)] = zero16
            return c

        lax.fori_loop(0, 288 // 16, occ_zero, 0)

        def scan_vec(i, base_vec):
            u0 = uids_v[pl.ds(i * 32, 16)]
            u1 = uids_v[pl.ds(i * 32 + 16, 16)]
            m0 = (u0 >= mylo) & (u0 < myhi)
            m1 = (u1 >= mylo) & (u1 < myhi)
            c0 = plsc.all_reduce_population_count(m0)
            c1 = plsc.all_reduce_population_count(m1)
            s0 = base_vec + plsc.cumsum(m0.astype(jnp.int32)) - 1
            s1 = base_vec + c0 + plsc.cumsum(m1.astype(jnp.int32)) - 1
            p0 = ((u0 - mylo) << 14) | (iota16 + i * 32)
            p1 = ((u1 - mylo) << 14) | (iota16 + i * 32 + 16)
            plsc.store_scatter(uids_v, [s0], p0, mask=m0)
            plsc.store_scatter(uids_v, [s1], p1, mask=m1)
            return base_vec + c0 + c1

        base_vec = lax.fori_loop(0, B // 32, scan_vec,
                                 jnp.zeros((16,), jnp.int32))
        count = base_vec[0]
        npair = (count + 31) // 32
        plsc.store_scatter(uids_v, [count + iota16], sentinel,
                           mask=(count + iota16) < B)
        plsc.store_scatter(uids_v, [count + 16 + iota16], sentinel,
                           mask=(count + 16 + iota16) < B)

        dmy = mylo - start
        ones16 = jnp.full((16,), 1, jnp.int32)

        def occ_build(j, c):
            w0 = uids_v[pl.ds(j * 32, 16)]
            w1 = uids_v[pl.ds(j * 32 + 16, 16)]
            r0 = w0 >> 14
            r1 = w1 >> 14
            plsc.store_scatter(occ_v, [(r0 + dmy) >> 7], ones16,
                               mask=r0 < RANGE)
            plsc.store_scatter(occ_v, [(r1 + dmy) >> 7], ones16,
                               mask=r1 < RANGE)
            return c

        lax.fori_loop(0, npair, occ_build, 0)

        def extract_half(src_ref, base_r, w, rel, m, sbh):
            pv = w & 16383
            loc = jnp.where(m, rel - base_r, 0)
            slots = sbh + plsc.cumsum(m.astype(jnp.int32)) - 1
            plsc.store_scatter(pos_s, [slots], pv, mask=m)
            for d in range(D):
                dvec = jnp.full((16,), d, jnp.int32)
                vals = plsc.load_gather(src_ref, [dvec, loc], mask=m)
                plsc.store_scatter(rows_v, [slots, dvec], vals, mask=m)

        def make_vec_body(src_ref, lo_r, hi_r, base_r):
            def vec_body(j, sb):
                w0 = uids_v[pl.ds(j * 32, 16)]
                w1 = uids_v[pl.ds(j * 32 + 16, 16)]
                rel0 = w0 >> 14
                rel1 = w1 >> 14
                m0 = (rel0 >= lo_r) & (rel0 < hi_r)
                m1 = (rel1 >= lo_r) & (rel1 < hi_r)
                c0 = plsc.all_reduce_population_count(m0)
                c1 = plsc.all_reduce_population_count(m1)
                tot = c0[0] + c1[0]
                do_flush = (sb[0] + tot) > CAP

                @pl.when(do_flush)
                def _flush():
                    pltpu.async_copy(
                        rows_v,
                        out_hbm.at[plsc.Indices(pos_s, ignored_value=-1)],
                        sem_o,
                    ).wait()
                    for t in range(CAP // 16):
                        pos_s[pl.ds(t * 16, 16)] = neg1

                sb = jnp.where(do_flush, 0, sb)

                @pl.when(tot > 0)
                def _extract():
                    @pl.when(c0[0] > 0)
                    def _h0():
                        extract_half(src_ref, base_r, w0, rel0, m0, sb)

                    @pl.when(c1[0] > 0)
                    def _h1():
                        extract_half(src_ref, base_r, w1, rel1, m1, sb + c0)

                return sb + c0 + c1

            return vec_body

        def process(s, buf, sb):
            off = start + s * SLAB
            off_c = jnp.minimum(off, MAXOFF)
            hi_m = jnp.minimum(off + SLAB, LAST_LO)
            body = make_vec_body(buf, off - mylo, hi_m - mylo, off_c - mylo)
            return lax.fori_loop(0, npair, body, sb)

        def pair_body(s2, sb):
            wait_slab(2 * s2, slab_a, sem_a)
            sb = process(2 * s2, slab_a, sb)
            stage(2 * s2 + 2, slab_a, sem_a)
            wait_slab(2 * s2 + 1, slab_b, sem_b)
            sb = process(2 * s2 + 1, slab_b, sb)
            stage(2 * s2 + 3, slab_b, sem_b)
            return sb

        sbase_vec = lax.fori_loop(0, NSLAB // 2, pair_body,
                                  jnp.zeros((16,), jnp.int32))
        wait_slab(jnp.int32(NSLAB), slab_a, sem_a)
        wait_slab(jnp.int32(NSLAB + 1), slab_b, sem_b)

        tail_body = make_vec_body(tail_v, LAST_LO - mylo, V - mylo,
                                  LAST_LO - mylo)
        lax.fori_loop(0, npair, tail_body, sbase_vec)

        pltpu.async_copy(
            rows_v, out_hbm.at[plsc.Indices(pos_s, ignored_value=-1)], sem_o
        ).wait()

    return gather_sweep


def _tc_body(emb_ref, age_ref,
             Wa1, ba1, Wa2, ba2, Wa3, ba3,
             Wo1, bo1, Wo2, bo2, Wo3, bo3,
             Wg1, bg1, Wg2, bg2, Wg3, bg3,
             age_out, gen_out, occ_out, loss_ref):
    x = emb_ref[:, :D]

    def mlp(w1, b1, w2, b2, w3, b3):
        h = jnp.dot(x, w1[...], preferred_element_type=jnp.float32) + b1[...]
        h = jnp.where(h >= 0, h, NEG_SLOPE * h)
        h = jnp.dot(h, w2[...], preferred_element_type=jnp.float32) + b2[...]
        h = jnp.where(h >= 0, h, NEG_SLOPE * h)
        return jnp.dot(h, w3[...], preferred_element_type=jnp.float32) + b3[...]

    a = mlp(Wa1, ba1, Wa2, ba2, Wa3, ba3)
    g = mlp(Wg1, bg1, Wg2, bg2, Wg3, bg3)
    o = mlp(Wo1, bo1, Wo2, bo2, Wo3, bo3)
    age_out[...] = a
    gen_out[...] = g
    occ_out[...] = o

    m = jnp.max(a, axis=1, keepdims=True)
    lse = jnp.log(jnp.sum(jnp.exp(a - m), axis=1, keepdims=True)) + m
    lbl = age_ref[...]
    cols = lax.broadcasted_iota(jnp.int32, (BLK, 7), 1)
    true_logit = jnp.sum(jnp.where(cols == lbl, a, 0.0), axis=1, keepdims=True)
    part = jnp.sum(lse - true_logit)

    @pl.when(pl.program_id(0) == 0)
    def _init():
        loss_ref[0, 0] = 0.0

    loss_ref[0, 0] += part

    @pl.when(pl.program_id(0) == pl.num_programs(0) - 1)
    def _finish():
        loss_ref[0, 0] = loss_ref[0, 0] * (1.0 / B)


def _full(shape):
    return pl.BlockSpec(shape, lambda i: (0,) * len(shape))


def _tc_specs():
    in_specs = [
        pl.BlockSpec((BLK, 128), lambda i: (i, 0)),
        pl.BlockSpec((BLK, 1), lambda i: (i, 0)),
        _full((D, 32)), _full((1, 32)), _full((32, 32)), _full((1, 32)),
        _full((32, 7)), _full((1, 7)),
        _full((D, 32)), _full((1, 32)), _full((32, 32)), _full((1, 32)),
        _full((32, 21)), _full((1, 21)),
        _full((D, 32)), _full((1, 32)), _full((32, 32)), _full((1, 32)),
        _full((32, 1)), _full((1, 1)),
    ]
    out_specs = [
        pl.BlockSpec((BLK, 7), lambda i: (i, 0)),
        pl.BlockSpec((BLK, 1), lambda i: (i, 0)),
        pl.BlockSpec((BLK, 21), lambda i: (i, 0)),
        pl.BlockSpec((1, 1), lambda i: (0, 0), memory_space=pltpu.SMEM),
    ]
    out_shapes = [
        jax.ShapeDtypeStruct((B, 7), jnp.float32),
        jax.ShapeDtypeStruct((B, 1), jnp.float32),
        jax.ShapeDtypeStruct((B, 21), jnp.float32),
        jax.ShapeDtypeStruct((1, 1), jnp.float32),
    ]
    return in_specs, out_specs, out_shapes


def kernel(user, gender, occupation, age, embeddings,
           Wa1, ba1, Wa2, ba2, Wa3, ba3,
           Wo1, bo1, Wo2, bo2, Wo3, bo3,
           Wg1, bg1, Wg2, bg2, Wg3, bg3):
    tableT = embeddings.T
    emb = _make_sc_gather()(tableT, tableT[:, LAST_LO:],
                            user.astype(jnp.int32))

    in_specs, out_specs, out_shapes = _tc_specs()
    age2 = age.astype(jnp.int32).reshape(B, 1)
    age_pred, gender_pred, occupation_pred, loss2 = pl.pallas_call(
        _tc_body,
        grid=(B // BLK,),
        in_specs=in_specs,
        out_specs=out_specs,
        out_shape=out_shapes,
    )(emb, age2,
      Wa1, ba1.reshape(1, 32), Wa2, ba2.reshape(1, 32), Wa3, ba3.reshape(1, 7),
      Wo1, bo1.reshape(1, 32), Wo2, bo2.reshape(1, 32), Wo3, bo3.reshape(1, 21),
      Wg1, bg1.reshape(1, 32), Wg2, bg2.reshape(1, 32), Wg3, bg3.reshape(1, 1))
    return (loss2[0, 0], age_pred, gender_pred, occupation_pred)


# speedup vs baseline: 1.0467x; 1.0467x over previous
import functools

import jax
import jax.numpy as jnp
from jax import lax
from jax.experimental import pallas as pl
from jax.experimental.pallas import tpu as pltpu
from jax.experimental.pallas import tpu_sc as plsc

B = 16384
V = 1_000_000
D = 64
BLK = 2048
NEG_SLOPE = 0.01

NW = 32
RANGE = V // NW
SLAB = 768
NSLAB = 42
CAP = 48
LAST_LO = V - 64
MAXOFF = 999168


def _make_sc_gather():
    info = plsc.get_sparse_core_info()
    nc, ns = info.num_cores, info.num_subcores
    mesh = plsc.VectorSubcoreMesh(core_axis_name="c", subcore_axis_name="s")

    @functools.partial(
        pl.kernel,
        mesh=mesh,
        out_type=jax.ShapeDtypeStruct((B, 128), jnp.float32),
        scratch_types=[
            pltpu.VMEM((B,), jnp.int32),
            pltpu.VMEM((D, SLAB), jnp.float32),
            pltpu.VMEM((D, SLAB), jnp.float32),
            pltpu.VMEM((D, 64), jnp.float32),
            pltpu.VMEM((CAP, 128), jnp.float32),
            pltpu.VMEM((CAP,), jnp.int32),
            pltpu.SemaphoreType.DMA,
            pltpu.SemaphoreType.DMA,
            pltpu.SemaphoreType.DMA,
        ],
        compiler_params=pltpu.CompilerParams(needs_layout_passes=False),
    )
    def gather_sweep(tableT_hbm, tail_hbm, idx_hbm, out_hbm,
                     uids_v, slab_a, slab_b, tail_v, rows_v, pos_s,
                     sem_a, sem_b, sem_o):
        wid = lax.axis_index("s") * nc + lax.axis_index("c")
        mylo = wid * RANGE
        myhi = mylo + RANGE
        start = (mylo // 128) * 128

        def stage(s, buf, sem_x):
            off = start + s * SLAB
            off_c = pl.multiple_of(jnp.minimum(off, MAXOFF), 128)
            pltpu.make_async_copy(
                tableT_hbm.at[:, pl.ds(off_c, SLAB)], buf, sem_x
            ).start()

        def wait_slab(buf, sem_x):
            pltpu.make_async_copy(
                tableT_hbm.at[:, pl.ds(0, SLAB)], buf, sem_x
            ).wait()

        stage(0, slab_a, sem_a)
        stage(1, slab_b, sem_b)
        pltpu.sync_copy(idx_hbm, uids_v)
        pltpu.sync_copy(tail_hbm, tail_v)

        neg1 = jnp.full((16,), -1, jnp.int32)
        sentinel = jnp.full((16,), 0x7FFFFFFF, jnp.int32)
        iota16 = lax.iota(jnp.int32, 16)

        def prefill_pos(i, c):
            pos_s[pl.ds(i * 16, 16)] = neg1
            return c

        lax.fori_loop(0, CAP // 16, prefill_pos, 0)


        def scan_vec(i, base_vec):
            u0 = uids_v[pl.ds(i * 32, 16)]
            u1 = uids_v[pl.ds(i * 32 + 16, 16)]
            m0 = (u0 >= mylo) & (u0 < myhi)
            m1 = (u1 >= mylo) & (u1 < myhi)
            c0 = plsc.all_reduce_population_count(m0)
            c1 = plsc.all_reduce_population_count(m1)
            s0 = base_vec + plsc.cumsum(m0.astype(jnp.int32)) - 1
            s1 = base_vec + c0 + plsc.cumsum(m1.astype(jnp.int32)) - 1
            p0 = ((u0 - mylo) << 14) | (iota16 + i * 32)
            p1 = ((u1 - mylo) << 14) | (iota16 + i * 32 + 16)
            plsc.store_scatter(uids_v, [s0], p0, mask=m0)
            plsc.store_scatter(uids_v, [s1], p1, mask=m1)
            return base_vec + c0 + c1

        base_vec = lax.fori_loop(0, B // 32, scan_vec,
                                 jnp.zeros((16,), jnp.int32))
        count = base_vec[0]
        npair = (count + 31) // 32
        plsc.store_scatter(uids_v, [count + iota16], sentinel,
                           mask=(count + iota16) < B)
        plsc.store_scatter(uids_v, [count + 16 + iota16], sentinel,
                           mask=(count + 16 + iota16) < B)


        def extract_half(src_ref, base_r, w, rel, m, sbh):
            pv = w & 16383
            loc = jnp.where(m, rel - base_r, 0)
            slots = sbh + plsc.cumsum(m.astype(jnp.int32)) - 1
            plsc.store_scatter(pos_s, [slots], pv, mask=m)
            for d in range(D):
                dvec = jnp.full((16,), d, jnp.int32)
                vals = plsc.load_gather(src_ref, [dvec, loc], mask=m)
                plsc.store_scatter(rows_v, [slots, dvec], vals, mask=m)

        def make_vec_body(src_ref, lo_r, hi_r, base_r):
            def vec_body(j, sb):
                w0 = uids_v[pl.ds(j * 32, 16)]
                w1 = uids_v[pl.ds(j * 32 + 16, 16)]
                rel0 = w0 >> 14
                rel1 = w1 >> 14
                m0 = (rel0 >= lo_r) & (rel0 < hi_r)
                m1 = (rel1 >= lo_r) & (rel1 < hi_r)
                c0 = plsc.all_reduce_population_count(m0)
                c1 = plsc.all_reduce_population_count(m1)
                tot = c0[0] + c1[0]
                do_flush = (sb[0] + tot) > CAP

                @pl.when(do_flush)
                def _flush():
                    pltpu.async_copy(
                        rows_v,
                        out_hbm.at[plsc.Indices(pos_s, ignored_value=-1)],
                        sem_o,
                    ).wait()
                    for t in range(CAP // 16):
                        pos_s[pl.ds(t * 16, 16)] = neg1

                sb = jnp.where(do_flush, 0, sb)

                @pl.when(tot > 0)
                def _extract():
                    @pl.when(c0[0] > 0)
                    def _h0():
                        extract_half(src_ref, base_r, w0, rel0, m0, sb)

                    @pl.when(c1[0] > 0)
                    def _h1():
                        extract_half(src_ref, base_r, w1, rel1, m1, sb + c0)

                return sb + c0 + c1

            return vec_body

        def process(s, buf, sb):
            off = start + s * SLAB
            off_c = jnp.minimum(off, MAXOFF)
            hi_m = jnp.minimum(off + SLAB, LAST_LO)
            body = make_vec_body(buf, off - mylo, hi_m - mylo, off_c - mylo)
            return lax.fori_loop(0, npair, body, sb)

        def pair_body(s2, sb):
            wait_slab(slab_a, sem_a)
            sb = process(2 * s2, slab_a, sb)
            stage(2 * s2 + 2, slab_a, sem_a)
            wait_slab(slab_b, sem_b)
            sb = process(2 * s2 + 1, slab_b, sb)
            stage(2 * s2 + 3, slab_b, sem_b)
            return sb

        sbase_vec = lax.fori_loop(0, NSLAB // 2, pair_body,
                                  jnp.zeros((16,), jnp.int32))
        wait_slab(slab_a, sem_a)
        wait_slab(slab_b, sem_b)

        tail_body = make_vec_body(tail_v, LAST_LO - mylo, V - mylo,
                                  LAST_LO - mylo)
        lax.fori_loop(0, npair, tail_body, sbase_vec)

        pltpu.async_copy(
            rows_v, out_hbm.at[plsc.Indices(pos_s, ignored_value=-1)], sem_o
        ).wait()

    return gather_sweep


def _tc_body(emb_ref, age_ref,
             Wa1, ba1, Wa2, ba2, Wa3, ba3,
             Wo1, bo1, Wo2, bo2, Wo3, bo3,
             Wg1, bg1, Wg2, bg2, Wg3, bg3,
             age_out, gen_out, occ_out, loss_ref):
    x = emb_ref[:, :D]

    def mlp(w1, b1, w2, b2, w3, b3):
        h = jnp.dot(x, w1[...], preferred_element_type=jnp.float32) + b1[...]
        h = jnp.where(h >= 0, h, NEG_SLOPE * h)
        h = jnp.dot(h, w2[...], preferred_element_type=jnp.float32) + b2[...]
        h = jnp.where(h >= 0, h, NEG_SLOPE * h)
        return jnp.dot(h, w3[...], preferred_element_type=jnp.float32) + b3[...]

    a = mlp(Wa1, ba1, Wa2, ba2, Wa3, ba3)
    g = mlp(Wg1, bg1, Wg2, bg2, Wg3, bg3)
    o = mlp(Wo1, bo1, Wo2, bo2, Wo3, bo3)
    age_out[...] = a
    gen_out[...] = g
    occ_out[...] = o

    m = jnp.max(a, axis=1, keepdims=True)
    lse = jnp.log(jnp.sum(jnp.exp(a - m), axis=1, keepdims=True)) + m
    lbl = age_ref[...]
    cols = lax.broadcasted_iota(jnp.int32, (BLK, 7), 1)
    true_logit = jnp.sum(jnp.where(cols == lbl, a, 0.0), axis=1, keepdims=True)
    part = jnp.sum(lse - true_logit)

    @pl.when(pl.program_id(0) == 0)
    def _init():
        loss_ref[0, 0] = 0.0

    loss_ref[0, 0] += part

    @pl.when(pl.program_id(0) == pl.num_programs(0) - 1)
    def _finish():
        loss_ref[0, 0] = loss_ref[0, 0] * (1.0 / B)


def _full(shape):
    return pl.BlockSpec(shape, lambda i: (0,) * len(shape))


def _tc_specs():
    in_specs = [
        pl.BlockSpec((BLK, 128), lambda i: (i, 0)),
        pl.BlockSpec((BLK, 1), lambda i: (i, 0)),
        _full((D, 32)), _full((1, 32)), _full((32, 32)), _full((1, 32)),
        _full((32, 7)), _full((1, 7)),
        _full((D, 32)), _full((1, 32)), _full((32, 32)), _full((1, 32)),
        _full((32, 21)), _full((1, 21)),
        _full((D, 32)), _full((1, 32)), _full((32, 32)), _full((1, 32)),
        _full((32, 1)), _full((1, 1)),
    ]
    out_specs = [
        pl.BlockSpec((BLK, 7), lambda i: (i, 0)),
        pl.BlockSpec((BLK, 1), lambda i: (i, 0)),
        pl.BlockSpec((BLK, 21), lambda i: (i, 0)),
        pl.BlockSpec((1, 1), lambda i: (0, 0), memory_space=pltpu.SMEM),
    ]
    out_shapes = [
        jax.ShapeDtypeStruct((B, 7), jnp.float32),
        jax.ShapeDtypeStruct((B, 1), jnp.float32),
        jax.ShapeDtypeStruct((B, 21), jnp.float32),
        jax.ShapeDtypeStruct((1, 1), jnp.float32),
    ]
    return in_specs, out_specs, out_shapes


def kernel(user, gender, occupation, age, embeddings,
           Wa1, ba1, Wa2, ba2, Wa3, ba3,
           Wo1, bo1, Wo2, bo2, Wo3, bo3,
           Wg1, bg1, Wg2, bg2, Wg3, bg3):
    tableT = embeddings.T
    emb = _make_sc_gather()(tableT, tableT[:, LAST_LO:],
                            user.astype(jnp.int32))

    in_specs, out_specs, out_shapes = _tc_specs()
    age2 = age.astype(jnp.int32).reshape(B, 1)
    age_pred, gender_pred, occupation_pred, loss2 = pl.pallas_call(
        _tc_body,
        grid=(B // BLK,),
        in_specs=in_specs,
        out_specs=out_specs,
        out_shape=out_shapes,
    )(emb, age2,
      Wa1, ba1.reshape(1, 32), Wa2, ba2.reshape(1, 32), Wa3, ba3.reshape(1, 7),
      Wo1, bo1.reshape(1, 32), Wo2, bo2.reshape(1, 32), Wo3, bo3.reshape(1, 21),
      Wg1, bg1.reshape(1, 32), Wg2, bg2.reshape(1, 32), Wg3, bg3.reshape(1, 1))
    return (loss2[0, 0], age_pred, gender_pred, occupation_pred)


# speedup vs baseline: 1.0510x; 1.0041x over previous
import functools

import jax
import jax.numpy as jnp
from jax import lax
from jax.experimental import pallas as pl
from jax.experimental.pallas import tpu as pltpu
from jax.experimental.pallas import tpu_sc as plsc

B = 16384
V = 1_000_000
D = 64
BLK = 4096
NEG_SLOPE = 0.01

NW = 32
RANGE = V // NW
SLAB = 768
NSLAB = 42
CAP = 48
LAST_LO = V - 64
MAXOFF = 999168


def _make_sc_gather():
    info = plsc.get_sparse_core_info()
    nc, ns = info.num_cores, info.num_subcores
    mesh = plsc.VectorSubcoreMesh(core_axis_name="c", subcore_axis_name="s")

    @functools.partial(
        pl.kernel,
        mesh=mesh,
        out_type=jax.ShapeDtypeStruct((B, 128), jnp.float32),
        scratch_types=[
            pltpu.VMEM((B,), jnp.int32),
            pltpu.VMEM((D, SLAB), jnp.float32),
            pltpu.VMEM((D, SLAB), jnp.float32),
            pltpu.VMEM((D, 64), jnp.float32),
            pltpu.VMEM((CAP, 128), jnp.float32),
            pltpu.VMEM((CAP,), jnp.int32),
            pltpu.SemaphoreType.DMA,
            pltpu.SemaphoreType.DMA,
            pltpu.SemaphoreType.DMA,
        ],
        compiler_params=pltpu.CompilerParams(needs_layout_passes=False),
    )
    def gather_sweep(tableT_hbm, tail_hbm, idx_hbm, out_hbm,
                     uids_v, slab_a, slab_b, tail_v, rows_v, pos_s,
                     sem_a, sem_b, sem_o):
        wid = lax.axis_index("s") * nc + lax.axis_index("c")
        mylo = wid * RANGE
        myhi = mylo + RANGE
        start = (mylo // 128) * 128

        def stage(s, buf, sem_x):
            off = start + s * SLAB
            off_c = pl.multiple_of(jnp.minimum(off, MAXOFF), 128)
            pltpu.make_async_copy(
                tableT_hbm.at[:, pl.ds(off_c, SLAB)], buf, sem_x
            ).start()

        def wait_slab(buf, sem_x):
            pltpu.make_async_copy(
                tableT_hbm.at[:, pl.ds(0, SLAB)], buf, sem_x
            ).wait()

        stage(0, slab_a, sem_a)
        stage(1, slab_b, sem_b)
        pltpu.sync_copy(idx_hbm, uids_v)
        pltpu.sync_copy(tail_hbm, tail_v)

        neg1 = jnp.full((16,), -1, jnp.int32)
        sentinel = jnp.full((16,), 0x7FFFFFFF, jnp.int32)
        iota16 = lax.iota(jnp.int32, 16)

        def prefill_pos(i, c):
            pos_s[pl.ds(i * 16, 16)] = neg1
            return c

        lax.fori_loop(0, CAP // 16, prefill_pos, 0)


        def scan_vec(i, base_vec):
            u0 = uids_v[pl.ds(i * 32, 16)]
            u1 = uids_v[pl.ds(i * 32 + 16, 16)]
            m0 = (u0 >= mylo) & (u0 < myhi)
            m1 = (u1 >= mylo) & (u1 < myhi)
            c0 = plsc.all_reduce_population_count(m0)
            c1 = plsc.all_reduce_population_count(m1)
            s0 = base_vec + plsc.cumsum(m0.astype(jnp.int32)) - 1
            s1 = base_vec + c0 + plsc.cumsum(m1.astype(jnp.int32)) - 1
            p0 = ((u0 - mylo) << 14) | (iota16 + i * 32)
            p1 = ((u1 - mylo) << 14) | (iota16 + i * 32 + 16)
            plsc.store_scatter(uids_v, [s0], p0, mask=m0)
            plsc.store_scatter(uids_v, [s1], p1, mask=m1)
            return base_vec + c0 + c1

        base_vec = lax.fori_loop(0, B // 32, scan_vec,
                                 jnp.zeros((16,), jnp.int32))
        count = base_vec[0]
        npair = (count + 31) // 32
        plsc.store_scatter(uids_v, [count + iota16], sentinel,
                           mask=(count + iota16) < B)
        plsc.store_scatter(uids_v, [count + 16 + iota16], sentinel,
                           mask=(count + 16 + iota16) < B)


        def extract_half(src_ref, base_r, w, rel, m, sbh):
            pv = w & 16383
            loc = jnp.where(m, rel - base_r, 0)
            slots = sbh + plsc.cumsum(m.astype(jnp.int32)) - 1
            plsc.store_scatter(pos_s, [slots], pv, mask=m)
            for d in range(D):
                dvec = jnp.full((16,), d, jnp.int32)
                vals = plsc.load_gather(src_ref, [dvec, loc], mask=m)
                plsc.store_scatter(rows_v, [slots, dvec], vals, mask=m)

        def make_vec_body(src_ref, lo_r, hi_r, base_r):
            def vec_body(j, sb):
                w0 = uids_v[pl.ds(j * 32, 16)]
                w1 = uids_v[pl.ds(j * 32 + 16, 16)]
                rel0 = w0 >> 14
                rel1 = w1 >> 14
                m0 = (rel0 >= lo_r) & (rel0 < hi_r)
                m1 = (rel1 >= lo_r) & (rel1 < hi_r)
                c0 = plsc.all_reduce_population_count(m0)
                c1 = plsc.all_reduce_population_count(m1)
                tot = c0[0] + c1[0]
                do_flush = (sb[0] + tot) > CAP

                @pl.when(do_flush)
                def _flush():
                    pltpu.async_copy(
                        rows_v,
                        out_hbm.at[plsc.Indices(pos_s, ignored_value=-1)],
                        sem_o,
                    ).wait()
                    for t in range(CAP // 16):
                        pos_s[pl.ds(t * 16, 16)] = neg1

                sb = jnp.where(do_flush, 0, sb)

                @pl.when(tot > 0)
                def _extract():
                    @pl.when(c0[0] > 0)
                    def _h0():
                        extract_half(src_ref, base_r, w0, rel0, m0, sb)

                    @pl.when(c1[0] > 0)
                    def _h1():
                        extract_half(src_ref, base_r, w1, rel1, m1, sb + c0)

                return sb + c0 + c1

            return vec_body

        def process(s, buf, sb):
            off = start + s * SLAB
            off_c = jnp.minimum(off, MAXOFF)
            hi_m = jnp.minimum(off + SLAB, LAST_LO)
            body = make_vec_body(buf, off - mylo, hi_m - mylo, off_c - mylo)
            return lax.fori_loop(0, npair, body, sb)

        def pair_body(s2, sb):
            wait_slab(slab_a, sem_a)
            sb = process(2 * s2, slab_a, sb)
            stage(2 * s2 + 2, slab_a, sem_a)
            wait_slab(slab_b, sem_b)
            sb = process(2 * s2 + 1, slab_b, sb)
            stage(2 * s2 + 3, slab_b, sem_b)
            return sb

        sbase_vec = lax.fori_loop(0, NSLAB // 2, pair_body,
                                  jnp.zeros((16,), jnp.int32))
        wait_slab(slab_a, sem_a)
        wait_slab(slab_b, sem_b)

        tail_body = make_vec_body(tail_v, LAST_LO - mylo, V - mylo,
                                  LAST_LO - mylo)
        lax.fori_loop(0, npair, tail_body, sbase_vec)

        pltpu.async_copy(
            rows_v, out_hbm.at[plsc.Indices(pos_s, ignored_value=-1)], sem_o
        ).wait()

    return gather_sweep


def _tc_body(emb_ref, age_ref,
             Wa1, ba1, Wa2, ba2, Wa3, ba3,
             Wo1, bo1, Wo2, bo2, Wo3, bo3,
             Wg1, bg1, Wg2, bg2, Wg3, bg3,
             age_out, gen_out, occ_out, loss_ref):
    x = emb_ref[:, :D]

    def mlp(w1, b1, w2, b2, w3, b3):
        h = jnp.dot(x, w1[...], preferred_element_type=jnp.float32) + b1[...]
        h = jnp.where(h >= 0, h, NEG_SLOPE * h)
        h = jnp.dot(h, w2[...], preferred_element_type=jnp.float32) + b2[...]
        h = jnp.where(h >= 0, h, NEG_SLOPE * h)
        return jnp.dot(h, w3[...], preferred_element_type=jnp.float32) + b3[...]

    a = mlp(Wa1, ba1, Wa2, ba2, Wa3, ba3)
    g = mlp(Wg1, bg1, Wg2, bg2, Wg3, bg3)
    o = mlp(Wo1, bo1, Wo2, bo2, Wo3, bo3)
    age_out[...] = a
    gen_out[...] = g
    occ_out[...] = o

    m = jnp.max(a, axis=1, keepdims=True)
    lse = jnp.log(jnp.sum(jnp.exp(a - m), axis=1, keepdims=True)) + m
    lbl = age_ref[...]
    cols = lax.broadcasted_iota(jnp.int32, (BLK, 7), 1)
    true_logit = jnp.sum(jnp.where(cols == lbl, a, 0.0), axis=1, keepdims=True)
    part = jnp.sum(lse - true_logit)

    @pl.when(pl.program_id(0) == 0)
    def _init():
        loss_ref[0, 0] = 0.0

    loss_ref[0, 0] += part

    @pl.when(pl.program_id(0) == pl.num_programs(0) - 1)
    def _finish():
        loss_ref[0, 0] = loss_ref[0, 0] * (1.0 / B)


def _full(shape):
    return pl.BlockSpec(shape, lambda i: (0,) * len(shape))


def _tc_specs():
    in_specs = [
        pl.BlockSpec((BLK, 128), lambda i: (i, 0)),
        pl.BlockSpec((BLK, 1), lambda i: (i, 0)),
        _full((D, 32)), _full((1, 32)), _full((32, 32)), _full((1, 32)),
        _full((32, 7)), _full((1, 7)),
        _full((D, 32)), _full((1, 32)), _full((32, 32)), _full((1, 32)),
        _full((32, 21)), _full((1, 21)),
        _full((D, 32)), _full((1, 32)), _full((32, 32)), _full((1, 32)),
        _full((32, 1)), _full((1, 1)),
    ]
    out_specs = [
        pl.BlockSpec((BLK, 7), lambda i: (i, 0)),
        pl.BlockSpec((BLK, 1), lambda i: (i, 0)),
        pl.BlockSpec((BLK, 21), lambda i: (i, 0)),
        pl.BlockSpec((1, 1), lambda i: (0, 0), memory_space=pltpu.SMEM),
    ]
    out_shapes = [
        jax.ShapeDtypeStruct((B, 7), jnp.float32),
        jax.ShapeDtypeStruct((B, 1), jnp.float32),
        jax.ShapeDtypeStruct((B, 21), jnp.float32),
        jax.ShapeDtypeStruct((1, 1), jnp.float32),
    ]
    return in_specs, out_specs, out_shapes


def kernel(user, gender, occupation, age, embeddings,
           Wa1, ba1, Wa2, ba2, Wa3, ba3,
           Wo1, bo1, Wo2, bo2, Wo3, bo3,
           Wg1, bg1, Wg2, bg2, Wg3, bg3):
    tableT = embeddings.T
    emb = _make_sc_gather()(tableT, tableT[:, LAST_LO:],
                            user.astype(jnp.int32))

    in_specs, out_specs, out_shapes = _tc_specs()
    age2 = age.astype(jnp.int32).reshape(B, 1)
    age_pred, gender_pred, occupation_pred, loss2 = pl.pallas_call(
        _tc_body,
        grid=(B // BLK,),
        in_specs=in_specs,
        out_specs=out_specs,
        out_shape=out_shapes,
    )(emb, age2,
      Wa1, ba1.reshape(1, 32), Wa2, ba2.reshape(1, 32), Wa3, ba3.reshape(1, 7),
      Wo1, bo1.reshape(1, 32), Wo2, bo2.reshape(1, 32), Wo3, bo3.reshape(1, 21),
      Wg1, bg1.reshape(1, 32), Wg2, bg2.reshape(1, 32), Wg3, bg3.reshape(1, 1))
    return (loss2[0, 0], age_pred, gender_pred, occupation_pred)


# speedup vs baseline: 1.0632x; 1.0116x over previous
import functools

import jax
import jax.numpy as jnp
from jax import lax
from jax.experimental import pallas as pl
from jax.experimental.pallas import tpu as pltpu
from jax.experimental.pallas import tpu_sc as plsc

B = 16384
V = 1_000_000
D = 64
BLK = 4096
NEG_SLOPE = 0.01

NW = 32
RANGE = V // NW
SLAB = 768
NSLAB = 42
CAP = 64
LAST_LO = V - 64
MAXOFF = 999168


def _make_sc_gather():
    info = plsc.get_sparse_core_info()
    nc, ns = info.num_cores, info.num_subcores
    mesh = plsc.VectorSubcoreMesh(core_axis_name="c", subcore_axis_name="s")

    @functools.partial(
        pl.kernel,
        mesh=mesh,
        out_type=jax.ShapeDtypeStruct((B, 128), jnp.float32),
        scratch_types=[
            pltpu.VMEM((B,), jnp.int32),
            pltpu.VMEM((D, SLAB), jnp.float32),
            pltpu.VMEM((D, SLAB), jnp.float32),
            pltpu.VMEM((CAP, 128), jnp.float32),
            pltpu.VMEM((CAP,), jnp.int32),
            pltpu.SemaphoreType.DMA,
            pltpu.SemaphoreType.DMA,
            pltpu.SemaphoreType.DMA,
        ],
        compiler_params=pltpu.CompilerParams(needs_layout_passes=False),
    )
    def gather_sweep(tableT_hbm, tail_hbm, idx_hbm, out_hbm,
                     uids_v, slab_a, slab_b, rows_v, pos_s,
                     sem_a, sem_b, sem_o):
        wid = lax.axis_index("s") * nc + lax.axis_index("c")
        mylo = wid * RANGE
        myhi = mylo + RANGE
        start = (mylo // 128) * 128

        def stage(s, buf, sem_x):
            off = start + s * SLAB
            off_c = pl.multiple_of(jnp.minimum(off, MAXOFF), 128)
            pltpu.make_async_copy(
                tableT_hbm.at[:, pl.ds(off_c, SLAB)], buf, sem_x
            ).start()

        def wait_slab(buf, sem_x):
            pltpu.make_async_copy(
                tableT_hbm.at[:, pl.ds(0, SLAB)], buf, sem_x
            ).wait()

        stage(0, slab_a, sem_a)
        stage(1, slab_b, sem_b)
        pltpu.sync_copy(idx_hbm, uids_v)

        neg1 = jnp.full((16,), -1, jnp.int32)
        sentinel = jnp.full((16,), 0x7FFFFFFF, jnp.int32)
        iota16 = lax.iota(jnp.int32, 16)

        def prefill_pos(i, c):
            pos_s[pl.ds(i * 16, 16)] = neg1
            return c

        lax.fori_loop(0, CAP // 16, prefill_pos, 0)


        def scan_vec(i, base_vec):
            u0 = uids_v[pl.ds(i * 32, 16)]
            u1 = uids_v[pl.ds(i * 32 + 16, 16)]
            m0 = (u0 >= mylo) & (u0 < myhi)
            m1 = (u1 >= mylo) & (u1 < myhi)
            c0 = plsc.all_reduce_population_count(m0)
            c1 = plsc.all_reduce_population_count(m1)
            s0 = base_vec + plsc.cumsum(m0.astype(jnp.int32)) - 1
            s1 = base_vec + c0 + plsc.cumsum(m1.astype(jnp.int32)) - 1
            p0 = ((u0 - mylo) << 14) | (iota16 + i * 32)
            p1 = ((u1 - mylo) << 14) | (iota16 + i * 32 + 16)
            plsc.store_scatter(uids_v, [s0], p0, mask=m0)
            plsc.store_scatter(uids_v, [s1], p1, mask=m1)
            return base_vec + c0 + c1

        base_vec = lax.fori_loop(0, B // 32, scan_vec,
                                 jnp.zeros((16,), jnp.int32))
        count = base_vec[0]
        npair = (count + 31) // 32
        plsc.store_scatter(uids_v, [count + iota16], sentinel,
                           mask=(count + iota16) < B)
        plsc.store_scatter(uids_v, [count + 16 + iota16], sentinel,
                           mask=(count + 16 + iota16) < B)


        def extract_half(src_ref, base_r, w, rel, m, sbh):
            pv = w & 16383
            loc = jnp.where(m, rel - base_r, 0)
            slots = sbh + plsc.cumsum(m.astype(jnp.int32)) - 1
            plsc.store_scatter(pos_s, [slots], pv, mask=m)
            for d in range(D):
                dvec = jnp.full((16,), d, jnp.int32)
                vals = plsc.load_gather(src_ref, [dvec, loc], mask=m)
                plsc.store_scatter(rows_v, [slots, dvec], vals, mask=m)

        def make_vec_body(src_ref, lo_r, hi_r, base_r):
            def vec_body(j, sb):
                w0 = uids_v[pl.ds(j * 32, 16)]
                w1 = uids_v[pl.ds(j * 32 + 16, 16)]
                rel0 = w0 >> 14
                rel1 = w1 >> 14
                m0 = (rel0 >= lo_r) & (rel0 < hi_r)
                m1 = (rel1 >= lo_r) & (rel1 < hi_r)
                c0 = plsc.all_reduce_population_count(m0)
                c1 = plsc.all_reduce_population_count(m1)
                tot = c0[0] + c1[0]
                do_flush = (sb[0] + tot) > CAP

                @pl.when(do_flush)
                def _flush():
                    pltpu.async_copy(
                        rows_v,
                        out_hbm.at[plsc.Indices(pos_s, ignored_value=-1)],
                        sem_o,
                    ).wait()
                    for t in range(CAP // 16):
                        pos_s[pl.ds(t * 16, 16)] = neg1

                sb = jnp.where(do_flush, 0, sb)

                @pl.when(tot > 0)
                def _extract():
                    @pl.when(c0[0] > 0)
                    def _h0():
                        extract_half(src_ref, base_r, w0, rel0, m0, sb)

                    @pl.when(c1[0] > 0)
                    def _h1():
                        extract_half(src_ref, base_r, w1, rel1, m1, sb + c0)

                return sb + c0 + c1

            return vec_body

        def process(s, buf, sb):
            off = start + s * SLAB
            off_c = jnp.minimum(off, MAXOFF)
            hi_m = jnp.minimum(off + SLAB, LAST_LO)
            body = make_vec_body(buf, off - mylo, hi_m - mylo, off_c - mylo)
            return lax.fori_loop(0, npair, body, sb)

        def pair_body(s2, sb):
            wait_slab(slab_a, sem_a)
            sb = process(2 * s2, slab_a, sb)
            stage(2 * s2 + 2, slab_a, sem_a)
            wait_slab(slab_b, sem_b)
            sb = process(2 * s2 + 1, slab_b, sb)
            stage(2 * s2 + 3, slab_b, sem_b)
            return sb

        sbase_vec = lax.fori_loop(0, NSLAB // 2, pair_body,
                                  jnp.zeros((16,), jnp.int32))
        wait_slab(slab_a, sem_a)
        wait_slab(slab_b, sem_b)

        pltpu.sync_copy(tail_hbm, slab_a.at[:, pl.ds(0, 128)])
        tail_body = make_vec_body(slab_a, LAST_LO - mylo, V - mylo,
                                  LAST_LO - mylo)
        lax.fori_loop(0, npair, tail_body, sbase_vec)

        pltpu.async_copy(
            rows_v, out_hbm.at[plsc.Indices(pos_s, ignored_value=-1)], sem_o
        ).wait()

    return gather_sweep


def _tc_body(emb_ref, age_ref,
             Wa1, ba1, Wa2, ba2, Wa3, ba3,
             Wo1, bo1, Wo2, bo2, Wo3, bo3,
             Wg1, bg1, Wg2, bg2, Wg3, bg3,
             age_out, gen_out, occ_out, loss_ref):
    x = emb_ref[:, :D]

    def mlp(w1, b1, w2, b2, w3, b3):
        h = jnp.dot(x, w1[...], preferred_element_type=jnp.float32) + b1[...]
        h = jnp.where(h >= 0, h, NEG_SLOPE * h)
        h = jnp.dot(h, w2[...], preferred_element_type=jnp.float32) + b2[...]
        h = jnp.where(h >= 0, h, NEG_SLOPE * h)
        return jnp.dot(h, w3[...], preferred_element_type=jnp.float32) + b3[...]

    a = mlp(Wa1, ba1, Wa2, ba2, Wa3, ba3)
    g = mlp(Wg1, bg1, Wg2, bg2, Wg3, bg3)
    o = mlp(Wo1, bo1, Wo2, bo2, Wo3, bo3)
    age_out[...] = a
    gen_out[...] = g
    occ_out[...] = o

    m = jnp.max(a, axis=1, keepdims=True)
    lse = jnp.log(jnp.sum(jnp.exp(a - m), axis=1, keepdims=True)) + m
    lbl = age_ref[...]
    cols = lax.broadcasted_iota(jnp.int32, (BLK, 7), 1)
    true_logit = jnp.sum(jnp.where(cols == lbl, a, 0.0), axis=1, keepdims=True)
    part = jnp.sum(lse - true_logit)

    @pl.when(pl.program_id(0) == 0)
    def _init():
        loss_ref[0, 0] = 0.0

    loss_ref[0, 0] += part

    @pl.when(pl.program_id(0) == pl.num_programs(0) - 1)
    def _finish():
        loss_ref[0, 0] = loss_ref[0, 0] * (1.0 / B)


def _full(shape):
    return pl.BlockSpec(shape, lambda i: (0,) * len(shape))


def _tc_specs():
    in_specs = [
        pl.BlockSpec((BLK, 128), lambda i: (i, 0)),
        pl.BlockSpec((BLK, 1), lambda i: (i, 0)),
        _full((D, 32)), _full((1, 32)), _full((32, 32)), _full((1, 32)),
        _full((32, 7)), _full((1, 7)),
        _full((D, 32)), _full((1, 32)), _full((32, 32)), _full((1, 32)),
        _full((32, 21)), _full((1, 21)),
        _full((D, 32)), _full((1, 32)), _full((32, 32)), _full((1, 32)),
        _full((32, 1)), _full((1, 1)),
    ]
    out_specs = [
        pl.BlockSpec((BLK, 7), lambda i: (i, 0)),
        pl.BlockSpec((BLK, 1), lambda i: (i, 0)),
        pl.BlockSpec((BLK, 21), lambda i: (i, 0)),
        pl.BlockSpec((1, 1), lambda i: (0, 0), memory_space=pltpu.SMEM),
    ]
    out_shapes = [
        jax.ShapeDtypeStruct((B, 7), jnp.float32),
        jax.ShapeDtypeStruct((B, 1), jnp.float32),
        jax.ShapeDtypeStruct((B, 21), jnp.float32),
        jax.ShapeDtypeStruct((1, 1), jnp.float32),
    ]
    return in_specs, out_specs, out_shapes


def kernel(user, gender, occupation, age, embeddings,
           Wa1, ba1, Wa2, ba2, Wa3, ba3,
           Wo1, bo1, Wo2, bo2, Wo3, bo3,
           Wg1, bg1, Wg2, bg2, Wg3, bg3):
    tableT = embeddings.T
    tail = jnp.pad(tableT[:, LAST_LO:], ((0, 0), (0, 64)))
    emb = _make_sc_gather()(tableT, tail, user.astype(jnp.int32))

    in_specs, out_specs, out_shapes = _tc_specs()
    age2 = age.astype(jnp.int32).reshape(B, 1)
    age_pred, gender_pred, occupation_pred, loss2 = pl.pallas_call(
        _tc_body,
        grid=(B // BLK,),
        in_specs=in_specs,
        out_specs=out_specs,
        out_shape=out_shapes,
    )(emb, age2,
      Wa1, ba1.reshape(1, 32), Wa2, ba2.reshape(1, 32), Wa3, ba3.reshape(1, 7),
      Wo1, bo1.reshape(1, 32), Wo2, bo2.reshape(1, 32), Wo3, bo3.reshape(1, 21),
      Wg1, bg1.reshape(1, 32), Wg2, bg2.reshape(1, 32), Wg3, bg3.reshape(1, 1))
    return (loss2[0, 0], age_pred, gender_pred, occupation_pred)
